# baseline (speedup 1.0000x reference)
import functools

import jax
import jax.numpy as jnp
from jax import lax
from jax.experimental import pallas as pl
from jax.experimental.pallas import tpu as pltpu
from jax.experimental.pallas import tpu_sc as plsc

_B, _P, _L, _M = 4, 8, 1024, 2048
_N = _B * _P
_LAN = 16
_KL = 8

_mesh = plsc.VectorSubcoreMesh(
    core_axis_name="c", subcore_axis_name="s", num_cores=2, num_subcores=16
)


@functools.partial(
    pl.kernel,
    out_type=jax.ShapeDtypeStruct((_N, 4 * _LAN), jnp.float32),
    mesh=_mesh,
    scratch_types=[
        pltpu.VMEM((_M + 4 * _LAN,), jnp.float32),
        pltpu.VMEM((_M + 4 * _LAN,), jnp.float32),
        pltpu.VMEM((_L + 4 * _LAN,), jnp.float32),
        pltpu.VMEM((_L + 4 * _LAN,), jnp.float32),
        pltpu.VMEM((_M + 4 * _LAN,), jnp.float32),
        pltpu.VMEM((_M + 4 * _LAN,), jnp.float32),
        pltpu.VMEM((_L + 4 * _LAN,), jnp.float32),
        pltpu.VMEM((_L + 4 * _LAN,), jnp.float32),
        pltpu.VMEM((_N,), jnp.int32),
        pltpu.VMEM((_N,), jnp.int32),
        pltpu.VMEM((4 * _LAN,), jnp.float32),
        pltpu.SemaphoreType.DMA,
        pltpu.SemaphoreType.DMA,
    ],
    compiler_params=pltpu.CompilerParams(needs_layout_passes=False),
)
def _chamfer_sc(x0h, x1h, y0h, y1h, xlh, ylh, out,
                ya0, ya1, xa0, xa1, yb0, yb1, xb0, xb1, xls, yls, ost,
                semA, semB):
    k = lax.axis_index("s") * 2 + lax.axis_index("c")
    pltpu.sync_copy(xlh, xls)
    pltpu.sync_copy(ylh, yls)
    iot = lax.iota(jnp.int32, _LAN)
    big = jnp.full((_LAN,), 1.0e10, jnp.float32)
    zero = jnp.zeros((_LAN,), jnp.float32)
    fzero = jnp.float32(0.0)

    def get_len(ref, m):
        c16 = pl.multiple_of((m // _LAN) * _LAN, _LAN)
        return jnp.max(jnp.where((c16 + iot) == m, ref[pl.ds(c16, _LAN)], 0))

    bfly = [jnp.bitwise_xor(iot, sh) for sh in (8, 4, 2, 1)]

    def lanemin(v):
        for idx in bfly:
            v = jnp.minimum(v, v.at[idx].get(mode="promise_in_bounds"))
        return v[0]

    def issue(m, y0s, y1s, x0s, x1s, sem):
        xbase = pl.multiple_of(m * _L, 8)
        ybase = pl.multiple_of(m * _M, 8)
        c1 = pltpu.async_copy(y0h.at[pl.ds(ybase, _M)], y0s.at[pl.ds(0, _M)], sem)
        c2 = pltpu.async_copy(y1h.at[pl.ds(ybase, _M)], y1s.at[pl.ds(0, _M)], sem)
        c3 = pltpu.async_copy(x0h.at[pl.ds(xbase, _L)], x0s.at[pl.ds(0, _L)], sem)
        c4 = pltpu.async_copy(x1h.at[pl.ds(xbase, _L)], x1s.at[pl.ds(0, _L)], sem)
        return (c1, c2, c3, c4)

    def process(m, carry, y0s, y1s, x0s, x1s):
        sx0, sx1, sy0, sy1 = carry
        nx = get_len(xls, m)
        ny = get_len(yls, m)
        kx = (k + m) % _N
        ky = (k + m + _N // 2) % _N

        vb = pl.multiple_of(jnp.minimum((ny // _LAN) * _LAN, _M - 4 * _LAN), _LAN)
        for off in (0, _LAN, 2 * _LAN, 3 * _LAN):
            mym = (vb + off + iot) < ny
            y0s[pl.ds(vb + off, _LAN)] = jnp.where(mym, y0s[pl.ds(vb + off, _LAN)], 2.0e4)
            y1s[pl.ds(vb + off, _LAN)] = jnp.where(mym, y1s[pl.ds(vb + off, _LAN)], 2.0e4)
        wb = pl.multiple_of(jnp.minimum((nx // _LAN) * _LAN, _L - 4 * _LAN), _LAN)
        for off in (0, _LAN, 2 * _LAN, 3 * _LAN):
            mxm = (wb + off + iot) < nx
            x0s[pl.ds(wb + off, _LAN)] = jnp.where(mxm, x0s[pl.ds(wb + off, _LAN)], 1.0e4)
            x1s[pl.ds(wb + off, _LAN)] = jnp.where(mxm, x1s[pl.ds(wb + off, _LAN)], 1.0e4)

        ncy4 = (ny + 4 * _LAN - 1) // (4 * _LAN)
        ncx4 = (nx + 4 * _LAN - 1) // (4 * _LAN)

        r32 = (nx + _N - 1) // _N
        rl8 = ((r32 + _KL - 1) // _KL) * _KL
        lo = kx * rl8
        hi = jnp.minimum(lo + rl8, nx)
        nrows = jnp.maximum(hi - lo, 0)
        nblk = (nrows + _KL - 1) // _KL

        def rblk(b, s):
            rb = pl.multiple_of(lo + b * _KL, _KL)
            xv0 = x0s[pl.ds(rb, _LAN)]
            xv1 = x1s[pl.ds(rb, _LAN)]
            av = xv0 * xv0 + xv1 * xv1
            t0 = -2.0 * xv0
            t1 = -2.0 * xv1
            bc0 = [jnp.full((_LAN,), t0[i]) for i in range(_KL)]
            bc1 = [jnp.full((_LAN,), t1[i]) for i in range(_KL)]

            def mstep(mc, accs):
                nacc = list(accs)
                for half in (0, 1, 2, 3):
                    mb = pl.multiple_of(mc * 4 * _LAN + half * _LAN, _LAN)
                    v0 = y0s[pl.ds(mb, _LAN)]
                    v1 = y1s[pl.ds(mb, _LAN)]
                    wc = v0 * v0 + v1 * v1
                    for i in range(_KL):
                        g = wc + bc0[i] * v0
                        g = g + bc1[i] * v1
                        nacc[i] = jnp.minimum(nacc[i], g)
                return tuple(nacc)

            accs = lax.fori_loop(0, ncy4, mstep, (big,) * _KL)
            for i in range(_KL):
                rm = lanemin(accs[i]) + av[i]
                s = s + jnp.where(b * _KL + i < nrows, rm, fzero)
            return s

        partx = lax.fori_loop(0, nblk, rblk, fzero)

        c32 = (ny + _N - 1) // _N
        cm8 = ((c32 + _KL - 1) // _KL) * _KL
        clo = ky * cm8
        chi = jnp.minimum(clo + cm8, ny)
        ncols = jnp.maximum(chi - clo, 0)
        ncb = (ncols + _KL - 1) // _KL

        def cblk(b, s):
            cb = pl.multiple_of(clo + b * _KL, _KL)
            yv0 = y0s[pl.ds(cb, _LAN)]
            yv1 = y1s[pl.ds(cb, _LAN)]
            wv = yv0 * yv0 + yv1 * yv1
            t0 = -2.0 * yv0
            t1 = -2.0 * yv1
            bc0 = [jnp.full((_LAN,), t0[i]) for i in range(_KL)]
            bc1 = [jnp.full((_LAN,), t1[i]) for i in range(_KL)]

            def rstep(rc, accs):
                nacc = list(accs)
                for half in (0, 1, 2, 3):
                    rv = pl.multiple_of(rc * 4 * _LAN + half * _LAN, _LAN)
                    u0 = x0s[pl.ds(rv, _LAN)]
                    u1 = x1s[pl.ds(rv, _LAN)]
                    ac = u0 * u0 + u1 * u1
                    for i in range(_KL):
                        g = ac + bc0[i] * u0
                        g = g + bc1[i] * u1
                        nacc[i] = jnp.minimum(nacc[i], g)
                return tuple(nacc)

            accs = lax.fori_loop(0, ncx4, rstep, (big,) * _KL)
            for i in range(_KL):
                cmn = lanemin(accs[i]) + wv[i]
                s = s + jnp.where(b * _KL + i < ncols, cmn, fzero)
            return s

        party = lax.fori_loop(0, ncb, cblk, fzero)

        hit = iot == (m % _LAN)
        g0 = m < _LAN
        hit0 = jnp.logical_and(hit, g0)
        hit1 = jnp.logical_and(hit, jnp.logical_not(g0))
        pxb = jnp.full((_LAN,), partx)
        pyb = jnp.full((_LAN,), party)
        sx0 = jnp.where(hit0, pxb, sx0)
        sx1 = jnp.where(hit1, pxb, sx1)
        sy0 = jnp.where(hit0, pyb, sy0)
        sy1 = jnp.where(hit1, pyb, sy1)
        return (sx0, sx1, sy0, sy1)

    issue(0, ya0, ya1, xa0, xa1, semA)

    def wait_slot(y0s, y1s, x0s, x1s, sem):
        pltpu.make_async_copy(y0h.at[pl.ds(0, _M)], y0s.at[pl.ds(0, _M)], sem).wait()
        pltpu.make_async_copy(y1h.at[pl.ds(0, _M)], y1s.at[pl.ds(0, _M)], sem).wait()
        pltpu.make_async_copy(x0h.at[pl.ds(0, _L)], x0s.at[pl.ds(0, _L)], sem).wait()
        pltpu.make_async_copy(x1h.at[pl.ds(0, _L)], x1s.at[pl.ds(0, _L)], sem).wait()

    def pair_step2(mm, carry):
        m0 = mm * 2
        m1 = m0 + 1
        m2 = jnp.minimum(m0 + 2, _N - 1)
        issue(m1, yb0, yb1, xb0, xb1, semB)
        wait_slot(ya0, ya1, xa0, xa1, semA)
        carry = process(m0, carry, ya0, ya1, xa0, xa1)
        issue(m2, ya0, ya1, xa0, xa1, semA)
        wait_slot(yb0, yb1, xb0, xb1, semB)
        carry = process(m1, carry, yb0, yb1, xb0, xb1)
        return carry

    carry = lax.fori_loop(0, _N // 2, pair_step2, (zero, zero, zero, zero))
    wait_slot(ya0, ya1, xa0, xa1, semA)
    sx0, sx1, sy0, sy1 = carry

    ost[pl.ds(0, _LAN)] = sx0
    ost[pl.ds(_LAN, _LAN)] = sx1
    ost[pl.ds(2 * _LAN, _LAN)] = sy0
    ost[pl.ds(3 * _LAN, _LAN)] = sy1
    pltpu.sync_copy(ost, out.at[k])


def kernel(boundaries, edgemaps, boundary_lengths, edgemaps_len):
    bx = boundaries[..., 0].reshape(_N * _L)
    by = boundaries[..., 1].reshape(_N * _L)
    ex = edgemaps[..., 0].reshape(_N * _M)
    ey = edgemaps[..., 1].reshape(_N * _M)
    xl = boundary_lengths.reshape(_N).astype(jnp.int32)
    yl = edgemaps_len.reshape(_N).astype(jnp.int32)
    o = _chamfer_sc(bx, by, ex, ey, xl, yl)
    sx = o[:, : 2 * _LAN].sum(axis=0)
    sy = o[:, 2 * _LAN :].sum(axis=0)
    xlf = xl.astype(jnp.float32)
    ylf = yl.astype(jnp.float32)
    loss = sx / xlf + sy / ylf
    return loss.reshape(_B, _P).mean(axis=1) * 10.0
